# Initial kernel scaffold; baseline (speedup 1.0000x reference)
import jax
import jax.numpy as jnp
from jax.experimental import pallas as pl


def kernel(flat_node2pin_start, flat_node2pin, sorted_pin_map):
    raise NotImplementedError("write your pallas kernel here")


# speedup vs baseline: 574.4336x; 574.4336x over previous
import functools

import jax
import jax.numpy as jnp
from jax import lax
from jax.experimental import pallas as pl
from jax.experimental.pallas import tpu as pltpu
from jax.experimental.pallas import tpu_sc as plsc

NUM_NODES_C = 100000
NUM_PINS_C = 1600000
NC = 2
NS = 16
NW = NC * NS
NPT = NUM_NODES_C // NW
OF_LEN = 3152
OUT_STRIDE = 3136
CHUNK = 4096
GB = 128
KG = CHUNK // GB
BIG = 0x7FFFFFFF
START_PAD = 100024


def _body(start_hbm, flat_hbm, map_hbm, out_hbm, offs_v, pins_v, vals_v,
          outb_v, sem):
    wid = lax.axis_index("c") * NS + lax.axis_index("s")
    n0 = wid * NPT
    a0 = (n0 // 8) * 8
    sh = n0 - a0
    pltpu.sync_copy(start_hbm.at[pl.ds(a0, OF_LEN)], offs_v)

    def _sread(i):
        return offs_v[pl.ds(i, 16)][0]

    s0 = _sread(sh)
    s1 = _sread(sh + NPT)
    cb0 = (s0 // 8) * 8
    nchunks = jnp.maximum(1, (s1 - cb0 + CHUNK - 1) // CHUNK)

    lane = lax.iota(jnp.int32, 16)
    bigv = jnp.full((16,), BIG, jnp.int32)

    def vmin_range(a, b, cb, av, ap):
        def cond(st):
            return st[0] < b

        def body(st):
            base, av, ap = st
            off = base - cb
            v = vals_v[pl.ds(off, 16)]
            p = pins_v[pl.ds(off, 16)]
            v = jnp.where(lane < (b - base), v, BIG)
            upd = v < av
            return (base + 16, jnp.where(upd, v, av), jnp.where(upd, p, ap))

        _, av, ap = lax.while_loop(cond, body, (a, av, ap))
        return av, ap

    def chunk_step(ci, st):
        node, cval, cpin = st
        cb = cb0 + ci * CHUNK
        ce = jnp.minimum(cb + CHUNK, s1)
        pltpu.sync_copy(flat_hbm.at[pl.ds(cb, CHUNK)],
                        pins_v.at[pl.ds(0, CHUNK)])
        handles = [
            pltpu.async_copy(map_hbm.at[pins_v.at[pl.ds(g * GB, GB)]],
                             vals_v.at[pl.ds(g * GB, GB)], sem)
            for g in range(KG)
        ]
        for h in handles:
            h.wait()

        def ncond(st):
            node = st[0]
            nb = _sread(sh + jnp.minimum(node, NPT - 1) + 1)
            return (node < NPT) & (nb <= ce)

        def nbody(st):
            node, cval, cpin = st
            na = _sread(sh + node)
            nb = _sread(sh + node + 1)
            a = jnp.maximum(na, cb)
            av, ap = vmin_range(a, nb, cb, cval, cpin)
            mv = jnp.min(av)
            pin = jnp.min(jnp.where(av == mv, ap, BIG))
            res = jnp.where(nb > na, pin, 0)
            plsc.store_scatter(outb_v, [jnp.full((16,), node, jnp.int32)],
                               jnp.full((16,), res, jnp.int32),
                               mask=lane == 0)
            return (node + 1, bigv, bigv)

        node, cval, cpin = lax.while_loop(ncond, nbody, (node, cval, cpin))

        na = _sread(sh + jnp.minimum(node, NPT - 1))
        do_part = (node < NPT) & (na < ce)
        a = jnp.maximum(na, cb)
        bb = jnp.where(do_part, ce, a)
        cval, cpin = vmin_range(a, bb, cb, cval, cpin)
        return (node, cval, cpin)

    lax.fori_loop(0, nchunks, chunk_step, (jnp.int32(0), bigv, bigv))
    pltpu.sync_copy(outb_v, out_hbm.at[pl.ds(wid * OUT_STRIDE, OUT_STRIDE)])


@jax.jit
def kernel(flat_node2pin_start, flat_node2pin, sorted_pin_map):
    num_nodes = flat_node2pin_start.shape[0] - 1
    start_p = jnp.pad(flat_node2pin_start,
                      (0, START_PAD - flat_node2pin_start.shape[0]),
                      mode="edge")
    flat_p = jnp.pad(flat_node2pin, (0, CHUNK + 16))

    mesh = plsc.VectorSubcoreMesh(core_axis_name="c", subcore_axis_name="s")
    run = pl.kernel(
        _body,
        out_type=jax.ShapeDtypeStruct((NW * OUT_STRIDE,), jnp.int32),
        mesh=mesh,
        compiler_params=pltpu.CompilerParams(needs_layout_passes=False),
        scratch_types=[
            pltpu.VMEM((OF_LEN,), jnp.int32),
            pltpu.VMEM((CHUNK + 16,), jnp.int32),
            pltpu.VMEM((CHUNK + 16,), jnp.int32),
            pltpu.VMEM((OUT_STRIDE,), jnp.int32),
            pltpu.SemaphoreType.DMA,
        ],
    )
    out_raw = run(start_p, flat_p, sorted_pin_map)
    return out_raw.reshape(NW, OUT_STRIDE)[:, :NPT].reshape(-1)[:num_nodes]


# speedup vs baseline: 1144.8530x; 1.9930x over previous
import jax
import jax.numpy as jnp
from jax import lax
from jax.experimental import pallas as pl
from jax.experimental.pallas import tpu as pltpu
from jax.experimental.pallas import tpu_sc as plsc

NUM_NODES_C = 100000
NUM_PINS_C = 1600000
NC = 2
NS = 16
NW = NC * NS
NPT = NUM_NODES_C // NW
OF_LEN = 3152
OUT_STRIDE = 3136
NGRP = OUT_STRIDE // 16
CHUNK = 4096
GB = 128
KG = CHUNK // GB
BIG = 0x7FFFFFFF
START_PAD = 100024


def _body(start_hbm, flat_hbm, map_hbm, out_hbm, offs_v, pins_v, vals_v,
          outb_v, sem):
    wid = lax.axis_index("c") * NS + lax.axis_index("s")
    n0 = wid * NPT
    a0 = (n0 // 8) * 8
    sh = n0 - a0
    pltpu.sync_copy(start_hbm.at[pl.ds(a0, OF_LEN)], offs_v)

    def _sread(i):
        return offs_v[pl.ds(i, 16)][0]

    s0 = _sread(sh)
    s1 = _sread(sh + NPT)
    cb0 = (s0 // 8) * 8
    nchunks = jnp.maximum(1, (s1 - cb0 + CHUNK - 1) // CHUNK)

    bigv = jnp.full((16,), BIG, jnp.int32)

    def fold_group(lo, hi, cb, av, ap):
        tmax = jnp.max(jnp.maximum(hi - lo, 0))

        def tstep(t, st):
            av, ap = st
            idxg = lo + t
            m = idxg < hi
            idxl = jnp.where(m, idxg - cb, 0)
            v = plsc.load_gather(vals_v, [idxl])
            p = plsc.load_gather(pins_v, [idxl])
            v = jnp.where(m, v, BIG)
            upd = v < av
            return (jnp.where(upd, v, av), jnp.where(upd, p, ap))

        return lax.fori_loop(0, tmax, tstep, (av, ap))

    def group_bounds(grp):
        na = offs_v[pl.ds(sh + grp * 16, 16)]
        ne = offs_v[pl.ds(sh + grp * 16 + 1, 16)]
        return jnp.minimum(na, s1), jnp.minimum(ne, s1)

    def chunk_step(ci, st):
        grp, cval, cpin = st
        cb = cb0 + ci * CHUNK
        ce = jnp.minimum(cb + CHUNK, s1)
        pltpu.sync_copy(flat_hbm.at[pl.ds(cb, CHUNK)],
                        pins_v.at[pl.ds(0, CHUNK)])
        handles = [
            pltpu.async_copy(map_hbm.at[pins_v.at[pl.ds(g * GB, GB)]],
                             vals_v.at[pl.ds(g * GB, GB)], sem)
            for g in range(KG)
        ]
        for h in handles:
            h.wait()

        def gcond(st):
            grp = st[0]
            ge = jnp.minimum(_sread(sh + jnp.minimum(grp + 1, NGRP) * 16), s1)
            return (grp < NGRP) & (ge <= ce)

        def gbody(st):
            grp, av, ap = st
            na, ne = group_bounds(grp)
            lo = jnp.maximum(na, cb)
            av, ap = fold_group(lo, ne, cb, av, ap)
            res = jnp.where(ne > na, ap, 0)
            outb_v[pl.ds(grp * 16, 16)] = res
            return (grp + 1, bigv, bigv)

        grp, cval, cpin = lax.while_loop(gcond, gbody, (grp, cval, cpin))

        na, ne = group_bounds(jnp.minimum(grp, NGRP - 1))
        live = grp < NGRP
        lo = jnp.maximum(na, cb)
        hi = jnp.where(live, jnp.minimum(ne, ce), lo)
        cval, cpin = fold_group(lo, hi, cb, cval, cpin)
        return (grp, cval, cpin)

    lax.fori_loop(0, nchunks, chunk_step, (jnp.int32(0), bigv, bigv))
    pltpu.sync_copy(outb_v, out_hbm.at[pl.ds(wid * OUT_STRIDE, OUT_STRIDE)])


@jax.jit
def kernel(flat_node2pin_start, flat_node2pin, sorted_pin_map):
    num_nodes = flat_node2pin_start.shape[0] - 1
    start_p = jnp.pad(flat_node2pin_start,
                      (0, START_PAD - flat_node2pin_start.shape[0]),
                      mode="edge")
    flat_p = jnp.pad(flat_node2pin, (0, CHUNK + 16))

    mesh = plsc.VectorSubcoreMesh(core_axis_name="c", subcore_axis_name="s")
    run = pl.kernel(
        _body,
        out_type=jax.ShapeDtypeStruct((NW * OUT_STRIDE,), jnp.int32),
        mesh=mesh,
        compiler_params=pltpu.CompilerParams(needs_layout_passes=False),
        scratch_types=[
            pltpu.VMEM((OF_LEN,), jnp.int32),
            pltpu.VMEM((CHUNK + 16,), jnp.int32),
            pltpu.VMEM((CHUNK + 16,), jnp.int32),
            pltpu.VMEM((OUT_STRIDE,), jnp.int32),
            pltpu.SemaphoreType.DMA,
        ],
    )
    out_raw = run(start_p, flat_p, sorted_pin_map)
    return out_raw.reshape(NW, OUT_STRIDE)[:, :NPT].reshape(-1)[:num_nodes]
